# baseline (speedup 1.0000x reference)
import functools

import jax
import jax.numpy as jnp
import numpy as np
from jax import lax
from jax.experimental import pallas as pl
from jax.experimental.pallas import tpu as pltpu
from jax.experimental.pallas import tpu_sc as plsc

_EMBED = 64
_B = 4096
_L = 200
_HALF = 112
_LPAD = 2 * _HALF
_LBUF = 208
_NW = 32
_RPW = _B // _NW
_NEG = -3.0e38
_NBUF = 8

_mesh = plsc.VectorSubcoreMesh(core_axis_name="c", subcore_axis_name="s")

_GATHER_DNUMS = lax.GatherDimensionNumbers(
    offset_dims=(), collapsed_slice_dims=(0,), start_index_map=(0,))

_EV = np.arange(0, 32, 2)
_PERM_HALF = np.concatenate([_EV, _EV + 1, _EV + 32, _EV + 33])
_PERM = np.concatenate([_PERM_HALF, _PERM_HALF + _EMBED])


def _bcast_lane(vec, lane):
    return lax.gather(
        vec, jnp.full((16, 1), lane, jnp.int32), _GATHER_DNUMS,
        slice_sizes=(1,), mode=lax.GatherScatterMode.PROMISE_IN_BOUNDS)


def _row_compute(r, rows, idx_v, feat_v):
    zero = jnp.zeros((16,), jnp.float32)
    negbf = jnp.full((32,), _NEG, jnp.bfloat16)
    init = (zero, zero, zero, zero, negbf, negbf, jnp.zeros((16,), jnp.int32))

    def make_body(h, tbase):
        def body(c, carry):
            idxv = idx_v[r, h, pl.ds(c * 16, 16)]
            t0 = tbase + c * 16

            def loads(j):
                return (rows[t0 + j, pl.ds(0, 32)],
                        rows[t0 + j, pl.ds(32, 32)])

            def fast(carry):
                s0, s1, s2, s3, ma, mb_, cnt = carry
                sacc = [s0, s1, s2, s3]
                for j in range(16):
                    va, vb = loads(j)
                    ma = jnp.maximum(ma, va)
                    mb_ = jnp.maximum(mb_, vb)
                    a0, a1 = plsc.unpack(va,
                                         format=plsc.PackFormat.INTERLEAVED)
                    b0, b1 = plsc.unpack(vb,
                                         format=plsc.PackFormat.INTERLEAVED)
                    for s, v in enumerate((a0, a1, b0, b1)):
                        sacc[s] = sacc[s] + v
                return (*sacc, ma, mb_, cnt + jnp.full((16,), 16, jnp.int32))

            def slow(carry):
                s0, s1, s2, s3, ma, mb_, cnt = carry
                sacc = [s0, s1, s2, s3]
                valid = idxv != jnp.zeros((16,), jnp.int32)
                cnt = cnt + plsc.all_reduce_population_count(valid)
                mvec = jnp.where(valid, jnp.zeros((16,), jnp.float32),
                                 jnp.full((16,), _NEG, jnp.float32))
                for j in range(16):
                    mbc = _bcast_lane(mvec, j)
                    va, vb = loads(j)
                    a0, a1 = plsc.unpack(va,
                                         format=plsc.PackFormat.INTERLEAVED)
                    b0, b1 = plsc.unpack(vb,
                                         format=plsc.PackFormat.INTERLEAVED)
                    ma = jnp.maximum(ma, plsc.pack(
                        a0 + mbc, a1 + mbc,
                        format=plsc.PackFormat.INTERLEAVED))
                    mb_ = jnp.maximum(mb_, plsc.pack(
                        b0 + mbc, b1 + mbc,
                        format=plsc.PackFormat.INTERLEAVED))
                    for s, v in enumerate((a0, a1, b0, b1)):
                        sacc[s] = sacc[s] + v
                return (*sacc, ma, mb_, cnt)

            return lax.cond(jnp.min(idxv) != 0, fast, slow, carry)
        return body

    carry = lax.fori_loop(0, _HALF // 16, make_body(0, 0), init)
    carry = lax.fori_loop(0, (_LBUF - _HALF) // 16, make_body(1, _HALF), carry)

    flen = jnp.maximum(carry[6], jnp.ones((16,), jnp.int32)).astype(jnp.float32)
    zvec = jnp.zeros((16,), jnp.float32)
    thresh = jnp.full((16,), -1.0e38, jnp.float32)
    for s in range(4):
        feat_v[r, pl.ds(s * 16, 16)] = carry[s] / flen
    mxs = (plsc.unpack(carry[4], format=plsc.PackFormat.INTERLEAVED)
           + plsc.unpack(carry[5], format=plsc.PackFormat.INTERLEAVED))
    for s in range(4):
        mx = mxs[s]
        feat_v[r, pl.ds(_EMBED + s * 16, 16)] = jnp.where(mx <= thresh, zvec, mx)


@functools.partial(
    pl.kernel,
    out_type=jax.ShapeDtypeStruct((_B, 2 * _EMBED), jnp.float32),
    mesh=_mesh,
    scratch_types=[
        pltpu.VMEM((_RPW, 2, _HALF), jnp.int32),
        [pltpu.VMEM((_LBUF, _EMBED), jnp.bfloat16) for _ in range(_NBUF)],
        pltpu.VMEM((_RPW, 2 * _EMBED), jnp.float32),
        [pltpu.SemaphoreType.DMA for _ in range(_NBUF)],
    ],
    compiler_params=pltpu.CompilerParams(use_tc_tiling_on_sc=False,
                                         needs_layout_passes=False),
)
def _pool_sc(x_hbm, table_hbm, feat_hbm, idx_v, rowbufs, feat_v, sems):
    wid = lax.axis_index("s") * 2 + lax.axis_index("c")
    base = wid * _RPW
    pltpu.sync_copy(x_hbm.at[pl.ds(base, _RPW)], idx_v)

    zbf = jnp.zeros((32,), jnp.bfloat16)
    for buf in rowbufs:
        for t in range(_L, _LBUF):
            buf[t, pl.ds(0, 32)] = zbf
            buf[t, pl.ds(32, 32)] = zbf

    def gather_start(r, rows, sem):
        pltpu.async_copy(table_hbm.at[idx_v.at[r, 0]],
                         rows.at[pl.ds(0, _HALF)], sem)
        pltpu.async_copy(table_hbm.at[idx_v.at[r, 1, pl.ds(0, _L - _HALF)]],
                         rows.at[pl.ds(_HALF, _L - _HALF)], sem)

    def gather_wait(rows, sem):
        pltpu.make_async_copy(table_hbm.at[pl.ds(0, _L)],
                              rows.at[pl.ds(0, _L)], sem).wait()

    for k in range(_NBUF):
        gather_start(k, rowbufs[k], sems[k])

    def g_body(g, carry):
        r0 = _NBUF * g
        for k in range(_NBUF):
            gather_wait(rowbufs[k], sems[k])
            _row_compute(r0 + k, rowbufs[k], idx_v, feat_v)

            @pl.when(g < _RPW // _NBUF - 1)
            def _():
                gather_start(r0 + k + _NBUF, rowbufs[k], sems[k])
        return carry

    lax.fori_loop(0, _RPW // _NBUF, g_body, 0)
    pltpu.sync_copy(feat_v, feat_hbm.at[pl.ds(base, _RPW)])


def _mlp_body(f_ref, wh_ref, bh_ref, wf_ref, bf_ref, o_ref):
    h = jnp.dot(f_ref[...], wh_ref[...], preferred_element_type=jnp.float32)
    h = jnp.maximum(h + bh_ref[...], 0.0)
    o_ref[...] = (jnp.dot(h, wf_ref[...], preferred_element_type=jnp.float32)
                  + bf_ref[...])


def kernel(x, table, Wh, bh, Wf, bf):
    x = x.astype(jnp.int32)
    xp = jnp.pad(x, ((0, 0), (0, _LPAD - _L))).reshape(_B, 2, _HALF)
    feat = _pool_sc(xp, table.astype(jnp.bfloat16))
    out = pl.pallas_call(
        _mlp_body,
        out_shape=jax.ShapeDtypeStruct((_B, Wf.shape[1]), jnp.float32),
    )(feat, Wh[_PERM, :], bh.reshape(1, -1), Wf, bf.reshape(1, -1))
    return out


# speedup vs baseline: 1.1382x; 1.1382x over previous
import functools

import jax
import jax.numpy as jnp
import numpy as np
from jax import lax
from jax.experimental import pallas as pl
from jax.experimental.pallas import tpu as pltpu
from jax.experimental.pallas import tpu_sc as plsc

_EMBED = 64
_B = 4096
_L = 200
_HALF = 112
_NCHUNK = 12
_NW = 32
_RPW = _B // _NW
_NEG = -3.0e38
_NBUF = 4

_mesh = plsc.VectorSubcoreMesh(core_axis_name="c", subcore_axis_name="s")

_GATHER_DNUMS = lax.GatherDimensionNumbers(
    offset_dims=(), collapsed_slice_dims=(0,), start_index_map=(0,))

_EV = np.arange(0, 32, 2)
_PERM_HALF = np.concatenate([_EV, _EV + 1, _EV + 32, _EV + 33])
_PERM = np.concatenate([_PERM_HALF, _PERM_HALF + _EMBED])


def _bcast_lane(vec, lane):
    return lax.gather(
        vec, jnp.full((16, 1), lane, jnp.int32), _GATHER_DNUMS,
        slice_sizes=(1,), mode=lax.GatherScatterMode.PROMISE_IN_BOUNDS)


def _unpack2(v):
    return plsc.unpack(v, format=plsc.PackFormat.INTERLEAVED)


def _row_compute(r, rows, idx_v, feat_v):
    zero = jnp.zeros((16,), jnp.float32)
    negbf = jnp.full((32,), _NEG, jnp.bfloat16)
    init = (zero, zero, zero, zero, negbf, negbf, jnp.zeros((16,), jnp.int32))
    ibase = r * _L

    def body(c, carry):
        idxv = idx_v[pl.ds(ibase + c * 16, 16)]
        t0 = c * 16

        def loads(j):
            return (rows[t0 + j, pl.ds(0, 32)], rows[t0 + j, pl.ds(32, 32)])

        def fast(carry):
            s0, s1, s2, s3, ma, mb_, cnt = carry
            sacc = [s0, s1, s2, s3]
            for j in range(16):
                va, vb = loads(j)
                ma = jnp.maximum(ma, va)
                mb_ = jnp.maximum(mb_, vb)
                a0, a1 = _unpack2(va)
                b0, b1 = _unpack2(vb)
                for s, v in enumerate((a0, a1, b0, b1)):
                    sacc[s] = sacc[s] + v
            return (*sacc, ma, mb_, cnt + jnp.full((16,), 16, jnp.int32))

        def slow(carry):
            s0, s1, s2, s3, ma, mb_, cnt = carry
            sacc = [s0, s1, s2, s3]
            valid = idxv != jnp.zeros((16,), jnp.int32)
            cnt = cnt + plsc.all_reduce_population_count(valid)
            mvec = jnp.where(valid, jnp.zeros((16,), jnp.float32),
                             jnp.full((16,), _NEG, jnp.float32))
            for j in range(16):
                mbc = _bcast_lane(mvec, j)
                va, vb = loads(j)
                a0, a1 = _unpack2(va)
                b0, b1 = _unpack2(vb)
                ma = jnp.maximum(ma, plsc.pack(
                    a0 + mbc, a1 + mbc, format=plsc.PackFormat.INTERLEAVED))
                mb_ = jnp.maximum(mb_, plsc.pack(
                    b0 + mbc, b1 + mbc, format=plsc.PackFormat.INTERLEAVED))
                for s, v in enumerate((a0, a1, b0, b1)):
                    sacc[s] = sacc[s] + v
            return (*sacc, ma, mb_, cnt)

        return lax.cond(jnp.min(idxv) != 0, fast, slow, carry)

    carry = lax.fori_loop(0, _NCHUNK, body, init)

    s0, s1, s2, s3, ma, mb_, cnt = carry
    sacc = [s0, s1, s2, s3]
    idxt = idx_v[pl.ds(ibase + 184, 16)]
    validt = jnp.logical_and(idxt != jnp.zeros((16,), jnp.int32),
                             lax.iota(jnp.int32, 16)
                             >= jnp.full((16,), 8, jnp.int32))
    cnt = cnt + plsc.all_reduce_population_count(validt)
    mvec = jnp.where(validt, jnp.zeros((16,), jnp.float32),
                     jnp.full((16,), _NEG, jnp.float32))
    for j in range(8, 16):
        mbc = _bcast_lane(mvec, j)
        t = 184 + j
        va = rows[t, pl.ds(0, 32)]
        vb = rows[t, pl.ds(32, 32)]
        a0, a1 = _unpack2(va)
        b0, b1 = _unpack2(vb)
        ma = jnp.maximum(ma, plsc.pack(
            a0 + mbc, a1 + mbc, format=plsc.PackFormat.INTERLEAVED))
        mb_ = jnp.maximum(mb_, plsc.pack(
            b0 + mbc, b1 + mbc, format=plsc.PackFormat.INTERLEAVED))
        for s, v in enumerate((a0, a1, b0, b1)):
            sacc[s] = sacc[s] + v

    flen = jnp.maximum(cnt, jnp.ones((16,), jnp.int32)).astype(jnp.float32)
    zvec = jnp.zeros((16,), jnp.float32)
    thresh = jnp.full((16,), -1.0e38, jnp.float32)
    for s in range(4):
        feat_v[r, pl.ds(s * 16, 16)] = sacc[s] / flen
    mxs = _unpack2(ma) + _unpack2(mb_)
    for s in range(4):
        mx = mxs[s]
        feat_v[r, pl.ds(_EMBED + s * 16, 16)] = jnp.where(mx <= thresh, zvec, mx)


@functools.partial(
    pl.kernel,
    out_type=jax.ShapeDtypeStruct((_B, 2 * _EMBED), jnp.float32),
    mesh=_mesh,
    scratch_types=[
        pltpu.VMEM((_RPW * _L,), jnp.int32),
        [pltpu.VMEM((_L, _EMBED), jnp.bfloat16) for _ in range(_NBUF)],
        pltpu.VMEM((_RPW, 2 * _EMBED), jnp.float32),
        [pltpu.SemaphoreType.DMA for _ in range(_NBUF)],
    ],
    compiler_params=pltpu.CompilerParams(use_tc_tiling_on_sc=False,
                                         needs_layout_passes=False),
)
def _pool_sc(x_hbm, table_hbm, feat_hbm, idx_v, rowbufs, feat_v, sems):
    wid = lax.axis_index("s") * 2 + lax.axis_index("c")
    base = wid * _RPW
    pltpu.sync_copy(x_hbm.at[pl.ds(base * _L, _RPW * _L)], idx_v)

    def gather_start(r, rows, sem):
        pltpu.async_copy(table_hbm.at[idx_v.at[pl.ds(r * _L, _HALF)]],
                         rows.at[pl.ds(0, _HALF)], sem)
        pltpu.async_copy(
            table_hbm.at[idx_v.at[pl.ds(r * _L + _HALF, _L - _HALF)]],
            rows.at[pl.ds(_HALF, _L - _HALF)], sem)

    def gather_wait(rows, sem):
        pltpu.make_async_copy(table_hbm.at[pl.ds(0, _L)], rows, sem).wait()

    for k in range(_NBUF):
        gather_start(k, rowbufs[k], sems[k])

    def g_body(g, carry):
        r0 = _NBUF * g
        for k in range(_NBUF):
            gather_wait(rowbufs[k], sems[k])
            _row_compute(r0 + k, rowbufs[k], idx_v, feat_v)

            @pl.when(g < _RPW // _NBUF - 1)
            def _():
                gather_start(r0 + k + _NBUF, rowbufs[k], sems[k])
        return carry

    lax.fori_loop(0, _RPW // _NBUF, g_body, 0)
    pltpu.sync_copy(feat_v, feat_hbm.at[pl.ds(base, _RPW)])


def _mlp_body(f_ref, wh_ref, bh_ref, wf_ref, bf_ref, o_ref):
    h = jnp.dot(f_ref[...], wh_ref[...], preferred_element_type=jnp.float32)
    h = jnp.maximum(h + bh_ref[...], 0.0)
    o_ref[...] = (jnp.dot(h, wf_ref[...], preferred_element_type=jnp.float32)
                  + bf_ref[...])


def kernel(x, table, Wh, bh, Wf, bf):
    xflat = x.astype(jnp.int32).reshape(-1)
    feat = _pool_sc(xflat, table.astype(jnp.bfloat16))
    out = pl.pallas_call(
        _mlp_body,
        out_shape=jax.ShapeDtypeStruct((_B, Wf.shape[1]), jnp.float32),
    )(feat, Wh[_PERM, :], bh.reshape(1, -1), Wf, bf.reshape(1, -1))
    return out
